# Initial kernel scaffold; baseline (speedup 1.0000x reference)
import jax
import jax.numpy as jnp
from jax.experimental import pallas as pl


def kernel(features, edge_index, W1, b1, W2, b2, W3, b3):
    raise NotImplementedError("write your pallas kernel here")


# speedup vs baseline: 1.2650x; 1.2650x over previous
import jax
import jax.numpy as jnp
from jax.experimental import pallas as pl

N = 10000
E = 320000


def _gc(x, W, b, src, dst, norm_out, norm_in):
    h = x * norm_out[:, None]
    h = h @ W
    msg = h[src]
    agg = jax.ops.segment_sum(msg, dst, num_segments=N)
    agg = agg * norm_in[:, None]
    return agg + b


def kernel(features, edge_index, W1, b1, W2, b2, W3, b3):
    src = edge_index[0]
    dst = edge_index[1]
    ones = jnp.ones((E,), dtype=jnp.float32)
    deg_out = jax.ops.segment_sum(ones, src, num_segments=N)
    deg_in = jax.ops.segment_sum(ones, dst, num_segments=N)
    norm_out = jnp.where(deg_out > 0, jax.lax.rsqrt(jnp.maximum(deg_out, 1.0)), 0.0)
    norm_in = jnp.where(deg_in > 0, jax.lax.rsqrt(jnp.maximum(deg_in, 1.0)), 0.0)
    x = features
    x = jax.nn.relu(_gc(x, W1, b1, src, dst, norm_out, norm_in))
    x = jax.nn.relu(_gc(x, W2, b2, src, dst, norm_out, norm_in))
    c = jax.ops.segment_sum(norm_in[dst], src, num_segments=N)
    cn = c * norm_out
    pooled = (cn @ x) @ W3 / N + b3
    return pooled.reshape(1, 1)


# speedup vs baseline: 6.3321x; 5.0054x over previous
import dataclasses
import functools

import jax
import jax.numpy as jnp
from jax import lax
from jax.experimental import pallas as pl
from jax.experimental.pallas import tpu as pltpu
from jax.experimental.pallas import tpu_sc as plsc

N = 10000
E = 320000
H = 128

NC = 2
NS = 16
NW = NC * NS
EPT = E // NW
C = 80
NCH = EPT // C
RPT = 624
TAIL = N - NS * RPT
ZR = 208

_MESH = plsc.VectorSubcoreMesh(
    core_axis_name="c", subcore_axis_name="s", num_cores=NC, num_subcores=NS
)

_SC_PARAMS = pltpu.CompilerParams()
if "needs_layout_passes" in pltpu.CompilerParams.__dataclass_fields__:
    _SC_PARAMS = dataclasses.replace(_SC_PARAMS, needs_layout_passes=False)


@functools.partial(
    pl.kernel,
    out_type=jax.ShapeDtypeStruct((NW, 2, N), jnp.float32),
    mesh=_MESH,
    scratch_types=[
        pltpu.VMEM((N,), jnp.float32),
        pltpu.VMEM((N,), jnp.float32),
        pltpu.VMEM((C,), jnp.int32),
        pltpu.VMEM((C,), jnp.int32),
    ],
    compiler_params=_SC_PARAMS,
)
def _deg_kernel(ei_hbm, out_hbm, dout_v, din_v, si_v, di_v):
    wid = lax.axis_index("s") * NC + lax.axis_index("c")

    @pl.loop(0, N, step=16)
    def _(i):
        z = jnp.zeros((16,), jnp.float32)
        dout_v[pl.ds(i, 16)] = z
        din_v[pl.ds(i, 16)] = z

    base0 = wid * EPT

    @pl.loop(0, EPT, step=C)
    def _(k):
        pltpu.sync_copy(ei_hbm.at[pl.ds(base0 + k, C)], si_v)
        pltpu.sync_copy(ei_hbm.at[pl.ds(E + base0 + k, C)], di_v)

        @pl.loop(0, C, step=16)
        def _(j):
            ones = jnp.ones((16,), jnp.float32)
            plsc.addupdate_scatter(dout_v, [si_v[pl.ds(j, 16)]], ones)
            plsc.addupdate_scatter(din_v, [di_v[pl.ds(j, 16)]], ones)

    pltpu.sync_copy(dout_v, out_hbm.at[wid, 0])
    pltpu.sync_copy(din_v, out_hbm.at[wid, 1])


@functools.partial(
    pl.kernel,
    out_type=jax.ShapeDtypeStruct((NW, N), jnp.float32),
    mesh=_MESH,
    scratch_types=[
        pltpu.VMEM((N,), jnp.float32),
        pltpu.VMEM((N,), jnp.float32),
        pltpu.VMEM((C,), jnp.int32),
        pltpu.VMEM((C,), jnp.int32),
    ],
    compiler_params=_SC_PARAMS,
)
def _cvec_kernel(ei_hbm, norms_hbm, out_hbm, nin_v, c_v, si_v, di_v):
    wid = lax.axis_index("s") * NC + lax.axis_index("c")
    pltpu.sync_copy(norms_hbm.at[1], nin_v)

    @pl.loop(0, N, step=16)
    def _(i):
        c_v[pl.ds(i, 16)] = jnp.zeros((16,), jnp.float32)

    base0 = wid * EPT

    @pl.loop(0, EPT, step=C)
    def _(k):
        pltpu.sync_copy(ei_hbm.at[pl.ds(base0 + k, C)], si_v)
        pltpu.sync_copy(ei_hbm.at[pl.ds(E + base0 + k, C)], di_v)

        @pl.loop(0, C, step=16)
        def _(j):
            vals = plsc.load_gather(nin_v, [di_v[pl.ds(j, 16)]])
            plsc.addupdate_scatter(c_v, [si_v[pl.ds(j, 16)]], vals)

    pltpu.sync_copy(c_v, out_hbm.at[wid])


@functools.partial(
    pl.kernel,
    out_type=jax.ShapeDtypeStruct((NC, N, H), jnp.float32),
    mesh=_MESH,
    scratch_types=[
        pltpu.VMEM_SHARED((N, H), jnp.float32),
        pltpu.VMEM((ZR, H), jnp.float32),
        pltpu.VMEM((C,), jnp.int32),
        pltpu.VMEM((C,), jnp.int32),
        pltpu.VMEM((C, H), jnp.float32),
        pltpu.SemaphoreType.DMA,
    ],
)
def _agg_kernel(h_hbm, ei_hbm, out_hbm, acc_s, zbuf_v, si_v, di_v, rows_v, sem):
    cid = lax.axis_index("c")
    sid = lax.axis_index("s")
    wid = sid * NC + cid

    @pl.loop(0, ZR)
    def _(r):
        @pl.loop(0, H, step=16)
        def _(h):
            zbuf_v[r, pl.ds(h, 16)] = jnp.zeros((16,), jnp.float32)

    @pl.loop(0, RPT, step=ZR)
    def _(z):
        pltpu.sync_copy(zbuf_v, acc_s.at[pl.ds(sid * RPT + z, ZR)])

    @pl.when(sid == NS - 1)
    def _():
        pltpu.sync_copy(zbuf_v.at[pl.ds(0, TAIL)], acc_s.at[pl.ds(NS * RPT, TAIL)])

    plsc.subcore_barrier()

    base0 = wid * EPT

    @pl.loop(0, EPT, step=C)
    def _(k):
        pltpu.sync_copy(ei_hbm.at[pl.ds(base0 + k, C)], si_v)
        pltpu.sync_copy(ei_hbm.at[pl.ds(E + base0 + k, C)], di_v)
        pltpu.async_copy(h_hbm.at[si_v], rows_v, sem).wait()
        pltpu.sync_copy(rows_v, acc_s.at[di_v], add=True)

    plsc.subcore_barrier()
    pltpu.sync_copy(
        acc_s.at[pl.ds(sid * RPT, RPT)], out_hbm.at[cid, pl.ds(sid * RPT, RPT)]
    )

    @pl.when(sid == NS - 1)
    def _():
        pltpu.sync_copy(
            acc_s.at[pl.ds(NS * RPT, TAIL)], out_hbm.at[cid, pl.ds(NS * RPT, TAIL)]
        )


def _prep_body(deg_ref, x_ref, w1_ref, norms_ref, h1_ref):
    deg = jnp.sum(deg_ref[...], axis=0)
    norms = jnp.where(deg > 0, lax.rsqrt(jnp.maximum(deg, 1.0)), 0.0)
    norms_ref[...] = norms
    h = x_ref[...] * norms[0][:, None]
    h1_ref[...] = jnp.dot(h, w1_ref[...], preferred_element_type=jnp.float32)


def _epi_body(agg_ref, norms_ref, b_ref, w_ref, h_ref):
    ni = norms_ref[1][:, None]
    x = jnp.maximum(
        (agg_ref[0] + agg_ref[1]) * ni + b_ref[...][None, :], 0.0
    )
    h_ref[...] = jnp.dot(
        x * norms_ref[0][:, None], w_ref[...], preferred_element_type=jnp.float32
    )


def _fin_body(agg_ref, norms_ref, b2_ref, cp_ref, w3_ref, b3_ref, out_ref):
    ni = norms_ref[1][:, None]
    x3 = jnp.maximum(
        (agg_ref[0] + agg_ref[1]) * ni + b2_ref[...][None, :], 0.0
    )
    c = jnp.sum(cp_ref[...], axis=0)
    cn = (c * norms_ref[0])[None, :]
    t = jnp.dot(cn, x3, preferred_element_type=jnp.float32)
    out_ref[...] = (
        jnp.dot(t, w3_ref[...], preferred_element_type=jnp.float32) / N
        + b3_ref[...][None, :]
    )


def kernel(features, edge_index, W1, b1, W2, b2, W3, b3):
    ei = edge_index.reshape(2 * E)

    degp = _deg_kernel(ei)

    norms, h1 = pl.pallas_call(
        _prep_body,
        out_shape=[
            jax.ShapeDtypeStruct((2, N), jnp.float32),
            jax.ShapeDtypeStruct((N, H), jnp.float32),
        ],
    )(degp, features, W1)

    cpart = _cvec_kernel(ei, norms)

    agg1 = _agg_kernel(h1, ei)

    h2 = pl.pallas_call(
        _epi_body,
        out_shape=jax.ShapeDtypeStruct((N, H), jnp.float32),
    )(agg1, norms, b1, W2)

    agg2 = _agg_kernel(h2, ei)

    pooled = pl.pallas_call(
        _fin_body,
        out_shape=jax.ShapeDtypeStruct((1, 1), jnp.float32),
    )(agg2, norms, b2, cpart, W3, b3)

    return pooled


# speedup vs baseline: 18.2418x; 2.8809x over previous
import dataclasses
import functools

import jax
import jax.numpy as jnp
from jax import lax
from jax.experimental import pallas as pl
from jax.experimental.pallas import tpu as pltpu
from jax.experimental.pallas import tpu_sc as plsc

N = 10000
E = 320000
H = 128

NC = 2
NS = 16
NW = NC * NS
EPT = E // NW
C = 80
NCH = EPT // C
RPT = 624
TAIL = N - NS * RPT
ZR = 208

_MESH = plsc.VectorSubcoreMesh(
    core_axis_name="c", subcore_axis_name="s", num_cores=NC, num_subcores=NS
)

_SC_PARAMS = pltpu.CompilerParams()
if "needs_layout_passes" in pltpu.CompilerParams.__dataclass_fields__:
    _SC_PARAMS = dataclasses.replace(_SC_PARAMS, needs_layout_passes=False)


@functools.partial(
    pl.kernel,
    out_type=jax.ShapeDtypeStruct((NW, 2, N), jnp.float32),
    mesh=_MESH,
    scratch_types=[
        pltpu.VMEM((N,), jnp.float32),
        pltpu.VMEM((N,), jnp.float32),
        pltpu.VMEM((NCH, C), jnp.int32),
        pltpu.VMEM((NCH, C), jnp.int32),
    ],
    compiler_params=_SC_PARAMS,
)
def _deg_kernel(ei_hbm, out_hbm, dout_v, din_v, si_v, di_v):
    wid = lax.axis_index("s") * NC + lax.axis_index("c")

    pltpu.sync_copy(ei_hbm.at[0, wid], si_v)
    pltpu.sync_copy(ei_hbm.at[1, wid], di_v)

    @pl.loop(0, N, step=16)
    def _(i):
        z = jnp.zeros((16,), jnp.float32)
        dout_v[pl.ds(i, 16)] = z
        din_v[pl.ds(i, 16)] = z

    @pl.loop(0, NCH)
    def _(k):
        @pl.loop(0, C, step=16)
        def _(j):
            ones = jnp.ones((16,), jnp.float32)
            plsc.addupdate_scatter(dout_v, [si_v[k, pl.ds(j, 16)]], ones)
            plsc.addupdate_scatter(din_v, [di_v[k, pl.ds(j, 16)]], ones)

    pltpu.sync_copy(dout_v, out_hbm.at[wid, 0])
    pltpu.sync_copy(din_v, out_hbm.at[wid, 1])


@functools.partial(
    pl.kernel,
    out_type=jax.ShapeDtypeStruct((NW, N), jnp.float32),
    mesh=_MESH,
    scratch_types=[
        pltpu.VMEM((N,), jnp.float32),
        pltpu.VMEM((N,), jnp.float32),
        pltpu.VMEM((NCH, C), jnp.int32),
        pltpu.VMEM((NCH, C), jnp.int32),
    ],
    compiler_params=_SC_PARAMS,
)
def _cvec_kernel(ei_hbm, norms_hbm, out_hbm, nin_v, c_v, si_v, di_v):
    wid = lax.axis_index("s") * NC + lax.axis_index("c")
    pltpu.sync_copy(norms_hbm.at[1], nin_v)
    pltpu.sync_copy(ei_hbm.at[0, wid], si_v)
    pltpu.sync_copy(ei_hbm.at[1, wid], di_v)

    @pl.loop(0, N, step=16)
    def _(i):
        c_v[pl.ds(i, 16)] = jnp.zeros((16,), jnp.float32)

    @pl.loop(0, NCH)
    def _(k):
        @pl.loop(0, C, step=16)
        def _(j):
            vals = plsc.load_gather(nin_v, [di_v[k, pl.ds(j, 16)]])
            plsc.addupdate_scatter(c_v, [si_v[k, pl.ds(j, 16)]], vals)

    pltpu.sync_copy(c_v, out_hbm.at[wid])


@functools.partial(
    pl.kernel,
    out_type=jax.ShapeDtypeStruct((NC, N, H), jnp.float32),
    mesh=_MESH,
    scratch_types=[
        pltpu.VMEM_SHARED((N, H), jnp.float32),
        pltpu.VMEM((EPT,), jnp.int32),
        pltpu.VMEM((NCH, C), jnp.int32),
        pltpu.VMEM((C, H), jnp.float32),
        pltpu.VMEM((C, H), jnp.float32),
        pltpu.SemaphoreType.DMA,
        pltpu.SemaphoreType.DMA,
    ],
)
def _agg_kernel(h_hbm, eis_hbm, eid_hbm, out_hbm, acc_s, si_v, di_v,
                rows_a, rows_b, sem_a, sem_b):
    cid = lax.axis_index("c")
    sid = lax.axis_index("s")
    wid = sid * NC + cid

    pltpu.sync_copy(eis_hbm.at[wid], si_v)
    pltpu.sync_copy(eid_hbm.at[wid], di_v)

    @pl.loop(0, C)
    def _(r):
        @pl.loop(0, H, step=16)
        def _(h):
            rows_a[r, pl.ds(h, 16)] = jnp.zeros((16,), jnp.float32)

    @pl.loop(0, RPT - C, step=C)
    def _(z):
        pltpu.sync_copy(rows_a, acc_s.at[pl.ds(sid * RPT + z, C)])

    pltpu.sync_copy(
        rows_a.at[pl.ds(0, RPT - 7 * C)],
        acc_s.at[pl.ds(sid * RPT + 7 * C, RPT - 7 * C)],
    )

    @pl.when(sid == NS - 1)
    def _():
        pltpu.sync_copy(rows_a.at[pl.ds(0, TAIL)], acc_s.at[pl.ds(NS * RPT, TAIL)])

    plsc.subcore_barrier()

    pltpu.async_copy(h_hbm.at[si_v.at[pl.ds(0, C)]], rows_a, sem_a)

    @pl.loop(0, NCH - 1, step=2)
    def _(k):
        pltpu.async_copy(h_hbm.at[si_v.at[pl.ds((k + 1) * C, C)]], rows_b, sem_b)
        pltpu.make_async_copy(h_hbm.at[si_v.at[pl.ds(k * C, C)]], rows_a, sem_a).wait()
        pltpu.sync_copy(rows_a, acc_s.at[di_v.at[k]], add=True)
        pltpu.async_copy(h_hbm.at[si_v.at[pl.ds((k + 2) * C, C)]], rows_a, sem_a)
        pltpu.make_async_copy(h_hbm.at[si_v.at[pl.ds((k + 1) * C, C)]], rows_b, sem_b).wait()
        pltpu.sync_copy(rows_b, acc_s.at[di_v.at[k + 1]], add=True)

    pltpu.make_async_copy(h_hbm.at[si_v.at[pl.ds((NCH - 1) * C, C)]], rows_a, sem_a).wait()
    pltpu.sync_copy(rows_a, acc_s.at[di_v.at[NCH - 1]], add=True)

    plsc.subcore_barrier()
    pltpu.sync_copy(
        acc_s.at[pl.ds(sid * RPT, RPT)], out_hbm.at[cid, pl.ds(sid * RPT, RPT)]
    )

    @pl.when(sid == NS - 1)
    def _():
        pltpu.sync_copy(
            acc_s.at[pl.ds(NS * RPT, TAIL)], out_hbm.at[cid, pl.ds(NS * RPT, TAIL)]
        )


def _prep_body(deg_ref, x_ref, w1_ref, norms_ref, h1_ref):
    deg = jnp.sum(deg_ref[...], axis=0)
    norms = jnp.where(deg > 0, lax.rsqrt(jnp.maximum(deg, 1.0)), 0.0)
    norms_ref[...] = norms
    h = x_ref[...] * norms[0][:, None]
    h1_ref[...] = jnp.dot(h, w1_ref[...], preferred_element_type=jnp.float32)


def _epi_body(agg_ref, norms_ref, b_ref, w_ref, h_ref):
    ni = norms_ref[1][:, None]
    x = jnp.maximum(
        (agg_ref[0] + agg_ref[1]) * ni + b_ref[...][None, :], 0.0
    )
    h_ref[...] = jnp.dot(
        x * norms_ref[0][:, None], w_ref[...], preferred_element_type=jnp.float32
    )


def _fin_body(agg_ref, norms_ref, b2_ref, cp_ref, w3_ref, b3_ref, out_ref):
    ni = norms_ref[1][:, None]
    x3 = jnp.maximum(
        (agg_ref[0] + agg_ref[1]) * ni + b2_ref[...][None, :], 0.0
    )
    c = jnp.sum(cp_ref[...], axis=0)
    cn = (c * norms_ref[0])[None, :]
    t = jnp.dot(cn, x3, preferred_element_type=jnp.float32)
    out_ref[...] = (
        jnp.dot(t, w3_ref[...], preferred_element_type=jnp.float32) / N
        + b3_ref[...][None, :]
    )


def kernel(features, edge_index, W1, b1, W2, b2, W3, b3):
    ei = edge_index.reshape(2, NW, NCH, C)
    eis = edge_index[0].reshape(NW, EPT)
    eid = edge_index[1].reshape(NW, NCH, C)

    degp = _deg_kernel(ei)

    norms, h1 = pl.pallas_call(
        _prep_body,
        out_shape=[
            jax.ShapeDtypeStruct((2, N), jnp.float32),
            jax.ShapeDtypeStruct((N, H), jnp.float32),
        ],
    )(degp, features, W1)

    cpart = _cvec_kernel(ei, norms)

    agg1 = _agg_kernel(h1, eis, eid)

    h2 = pl.pallas_call(
        _epi_body,
        out_shape=jax.ShapeDtypeStruct((N, H), jnp.float32),
    )(agg1, norms, b1, W2)

    agg2 = _agg_kernel(h2, eis, eid)

    pooled = pl.pallas_call(
        _fin_body,
        out_shape=jax.ShapeDtypeStruct((1, 1), jnp.float32),
    )(agg2, norms, b2, cpart, W3, b3)

    return pooled
